# baseline (speedup 1.0000x reference)
import functools

import jax
import jax.numpy as jnp
from jax.experimental import pallas as pl
from jax.experimental.pallas import tpu as pltpu

N_FIELDS = 26
BATCH = 1024
VOCAB = 1000
EMB = 16

BB = 256


def _mm_kernel(x_ref, w_ref, o_ref):
    x = x_ref[0].astype(jnp.bfloat16)
    w = w_ref[0].astype(jnp.bfloat16)
    z = jax.lax.dot_general(
        x, w, (((1,), (1,)), ((), ())),
        preferred_element_type=jnp.float32)
    o_ref[0] = z


@functools.partial(jax.jit)
def kernel(xs, W):
    F, B, V = xs.shape
    _, E, _ = W.shape
    nb = B // BB
    z = pl.pallas_call(
        _mm_kernel,
        grid=(F, nb),
        in_specs=[
            pl.BlockSpec((1, BB, V), lambda f, b: (f, b, 0)),
            pl.BlockSpec((1, E, V), lambda f, b: (f, 0, 0)),
        ],
        out_specs=pl.BlockSpec((1, BB, E), lambda f, b: (f, b, 0)),
        out_shape=jax.ShapeDtypeStruct((F, B, E), jnp.float32),
        compiler_params=pltpu.CompilerParams(
            dimension_semantics=("parallel", "parallel"),
        ),
    )(xs, W)
    return jnp.transpose(z, (1, 2, 0))
